# baseline (speedup 1.0000x reference)
import functools

import jax
import jax.numpy as jnp
from jax.experimental import pallas as pl
from jax.experimental.pallas import tpu as pltpu

N = 50000
E = 800000
G = 128
ND = 86
ED = 18
HID = 128
H = 2
C = 64
EH = 64


def _node_enc_body(x_ref, w_ref, b_ref, g_ref, be_ref, o_ref):
    h = jnp.dot(x_ref[...], w_ref[...], preferred_element_type=jnp.float32)
    h = h + b_ref[...]
    mu = jnp.mean(h, axis=-1, keepdims=True)
    v = jnp.mean((h - mu) * (h - mu), axis=-1, keepdims=True)
    h = (h - mu) * jax.lax.rsqrt(v + 1e-5) * g_ref[...] + be_ref[...]
    o_ref[...] = jnp.maximum(h, 0.0)


def _node_encoder(x, p):
    blk = 1000
    grid = (N // blk,)
    return pl.pallas_call(
        _node_enc_body,
        grid=grid,
        in_specs=[
            pl.BlockSpec((blk, ND), lambda i: (i, 0)),
            pl.BlockSpec((ND, HID), lambda i: (0, 0)),
            pl.BlockSpec((HID,), lambda i: (0,)),
            pl.BlockSpec((HID,), lambda i: (0,)),
            pl.BlockSpec((HID,), lambda i: (0,)),
        ],
        out_specs=pl.BlockSpec((blk, HID), lambda i: (i, 0)),
        out_shape=jax.ShapeDtypeStruct((N, HID), jnp.float32),
    )(x, p['W'], p['b'], p['g'], p['be'])


def _edge_enc_body(ea_ref, w_ref, b_ref, g_ref, be_ref, o_ref, sum_ref):
    h = jnp.dot(ea_ref[...], w_ref[...], preferred_element_type=jnp.float32)
    h = h + b_ref[...]
    mu = jnp.mean(h, axis=-1, keepdims=True)
    v = jnp.mean((h - mu) * (h - mu), axis=-1, keepdims=True)
    h = (h - mu) * jax.lax.rsqrt(v + 1e-5) * g_ref[...] + be_ref[...]
    h = jnp.maximum(h, 0.0)
    o_ref[...] = h

    @pl.when(pl.program_id(0) == 0)
    def _():
        sum_ref[...] = jnp.zeros_like(sum_ref)

    sum_ref[...] += jnp.sum(h, axis=0, keepdims=True)


def _edge_encoder(ea, p):
    blk = 2000
    grid = (E // blk,)
    out, colsum = pl.pallas_call(
        _edge_enc_body,
        grid=grid,
        in_specs=[
            pl.BlockSpec((blk, ED), lambda i: (i, 0)),
            pl.BlockSpec((ED, EH), lambda i: (0, 0)),
            pl.BlockSpec((EH,), lambda i: (0,)),
            pl.BlockSpec((EH,), lambda i: (0,)),
            pl.BlockSpec((EH,), lambda i: (0,)),
        ],
        out_specs=[
            pl.BlockSpec((blk, EH), lambda i: (i, 0)),
            pl.BlockSpec((1, EH), lambda i: (0, 0)),
        ],
        out_shape=[
            jax.ShapeDtypeStruct((E, EH), jnp.float32),
            jax.ShapeDtypeStruct((1, EH), jnp.float32),
        ],
    )(ea, p['W'], p['b'], p['g'], p['be'])
    return out, colsum / E


def _seg_softmax(a, seg, n):
    m = jax.ops.segment_max(a, seg, num_segments=n)
    e = jnp.exp(a - m[seg])
    s = jax.ops.segment_sum(e, seg, num_segments=n)
    return e / (s[seg] + 1e-16)


def _gatv2(x, edge_index, eattr, mean_e, p):
    n = x.shape[0]
    loop = jnp.arange(n, dtype=edge_index.dtype)
    src = jnp.concatenate([edge_index[0], loop])
    dst = jnp.concatenate([edge_index[1], loop])
    ea = jnp.concatenate([eattr, jnp.broadcast_to(mean_e, (n, eattr.shape[1]))], axis=0)
    xl = (x @ p['lin_l_W'] + p['lin_l_b']).reshape(n, H, C)
    xr = (x @ p['lin_r_W'] + p['lin_r_b']).reshape(n, H, C)
    ee = (ea @ p['lin_e_W']).reshape(-1, H, C)
    xj = xl[src]
    xi = xr[dst]
    m = jax.nn.leaky_relu(xj + xi + ee, negative_slope=0.2)
    alpha = jnp.sum(m * p['att'][None], axis=-1)
    alpha = _seg_softmax(alpha, dst, n)
    out = jax.ops.segment_sum(xj * alpha[..., None], dst, num_segments=n)
    return out.reshape(n, H * C) + p['bias']


def _ln(x, g, b):
    mu = jnp.mean(x, axis=-1, keepdims=True)
    v = jnp.var(x, axis=-1, keepdims=True)
    return (x - mu) / jnp.sqrt(v + 1e-5) * g + b


def _head(g, hp):
    h = g @ hp['W1'] + hp['b1']
    h = jax.nn.relu(_ln(h, hp['g'], hp['be']))
    h = jax.nn.relu(h @ hp['W2'] + hp['b2'])
    return h @ hp['W3'] + hp['b3']


def kernel(x, edge_attr, edge_index, batch, params):
    h = _node_encoder(x, params['node'])
    ea, mean_e = _edge_encoder(edge_attr, params['edge'])
    for lp in params['layers']:
        xn = _gatv2(h, edge_index, ea, mean_e, lp)
        xn = _ln(xn, lp['ng'], lp['nb'])
        h = jax.nn.relu(xn) + h
    counts = jax.ops.segment_sum(jnp.ones((h.shape[0],), jnp.float32), batch, num_segments=G)
    sums = jax.ops.segment_sum(h, batch, num_segments=G)
    gemb = sums / jnp.clip(counts, 1.0)[:, None]
    gemb = jnp.tanh(gemb @ params['readout']['W'] + params['readout']['b'])
    return tuple(_head(gemb, params['heads'][t]) for t in ['DAT', 'NET', 'SERT'])


# speedup vs baseline: 8.3540x; 8.3540x over previous
import functools

import jax
import jax.numpy as jnp
from jax import lax
from jax.experimental import pallas as pl
from jax.experimental.pallas import tpu as pltpu
from jax.experimental.pallas import tpu_sc as plsc

N = 50000
E = 800000
G = 128
ND = 86
ED = 18
HID = 128
EH = 64

NC = 2
NS = 16
NW = NC * NS
EPW = 25600
EPAD = NW * EPW
PADW = EPAD - E
NEG = -3e38

NQ = 4
CQ = HID // NQ
NP = 50048
RPT = NP // NS

WA = 128

_mesh = plsc.VectorSubcoreMesh(core_axis_name="c", subcore_axis_name="s")


def _wid():
    return lax.axis_index("s") * NC + lax.axis_index("c")


def _node_enc_body(x_ref, w_ref, b_ref, g_ref, be_ref, o_ref):
    h = jnp.dot(x_ref[...], w_ref[...], preferred_element_type=jnp.float32)
    h = h + b_ref[...]
    mu = jnp.mean(h, axis=-1, keepdims=True)
    v = jnp.mean((h - mu) * (h - mu), axis=-1, keepdims=True)
    h = (h - mu) * lax.rsqrt(v + 1e-5) * g_ref[...] + be_ref[...]
    o_ref[...] = jnp.maximum(h, 0.0)


def _node_encoder(x, p):
    blk = 1000
    return pl.pallas_call(
        _node_enc_body,
        grid=(N // blk,),
        in_specs=[
            pl.BlockSpec((blk, ND), lambda i: (i, 0)),
            pl.BlockSpec((ND, HID), lambda i: (0, 0)),
            pl.BlockSpec((HID,), lambda i: (0,)),
            pl.BlockSpec((HID,), lambda i: (0,)),
            pl.BlockSpec((HID,), lambda i: (0,)),
        ],
        out_specs=pl.BlockSpec((blk, HID), lambda i: (i, 0)),
        out_shape=jax.ShapeDtypeStruct((N, HID), jnp.float32),
    )(x, p['W'], p['b'], p['g'], p['be'])


def _edge_enc_body(ea_ref, w_ref, b_ref, g_ref, be_ref, o_ref, sum_ref):
    h = jnp.dot(ea_ref[...], w_ref[...], preferred_element_type=jnp.float32)
    h = h + b_ref[...]
    mu = jnp.mean(h, axis=-1, keepdims=True)
    v = jnp.mean((h - mu) * (h - mu), axis=-1, keepdims=True)
    h = (h - mu) * lax.rsqrt(v + 1e-5) * g_ref[...] + be_ref[...]
    h = jnp.maximum(h, 0.0)
    o_ref[...] = h

    @pl.when(pl.program_id(0) == 0)
    def _():
        sum_ref[...] = jnp.zeros_like(sum_ref)

    sum_ref[...] += jnp.sum(h, axis=0, keepdims=True)


def _edge_encoder(ea, p):
    blk = 2000
    out, colsum = pl.pallas_call(
        _edge_enc_body,
        grid=(E // blk,),
        in_specs=[
            pl.BlockSpec((blk, ED), lambda i: (i, 0)),
            pl.BlockSpec((ED, EH), lambda i: (0, 0)),
            pl.BlockSpec((EH,), lambda i: (0,)),
            pl.BlockSpec((EH,), lambda i: (0,)),
            pl.BlockSpec((EH,), lambda i: (0,)),
        ],
        out_specs=[
            pl.BlockSpec((blk, EH), lambda i: (i, 0)),
            pl.BlockSpec((1, EH), lambda i: (0, 0)),
        ],
        out_shape=[
            jax.ShapeDtypeStruct((E, EH), jnp.float32),
            jax.ShapeDtypeStruct((1, EH), jnp.float32),
        ],
    )(ea, p['W'], p['b'], p['g'], p['be'])
    return out, colsum / E


def _pre_body(h_ref, wl, bl, wr, br, we, me, att, xl_o, xr_o, al_o, amax_o):
    hb = h_ref[...]
    xl = jnp.dot(hb, wl[...], preferred_element_type=jnp.float32) + bl[...]
    xr = jnp.dot(hb, wr[...], preferred_element_type=jnp.float32) + br[...]
    eel = jnp.dot(me[...], we[...], preferred_element_type=jnp.float32)
    z = xl + xr + eel
    m = jnp.maximum(z, 0.2 * z) * att[...]
    a0 = jnp.sum(m[:, :64], axis=1, keepdims=True)
    a1 = jnp.sum(m[:, 64:], axis=1, keepdims=True)
    al = jnp.concatenate([a0, a1], axis=1)
    xl_o[...] = xl
    xr_o[...] = xr
    al_o[...] = al

    @pl.when(pl.program_id(0) == 0)
    def _():
        amax_o[...] = jnp.full_like(amax_o, NEG)

    amax_o[...] = jnp.maximum(amax_o[...], jnp.max(al))


def _pre_layer(h, mean_e, att1, lp):
    blk = 1000
    return pl.pallas_call(
        _pre_body,
        grid=(N // blk,),
        in_specs=[
            pl.BlockSpec((blk, HID), lambda i: (i, 0)),
            pl.BlockSpec((HID, HID), lambda i: (0, 0)),
            pl.BlockSpec((HID,), lambda i: (0,)),
            pl.BlockSpec((HID, HID), lambda i: (0, 0)),
            pl.BlockSpec((HID,), lambda i: (0,)),
            pl.BlockSpec((EH, HID), lambda i: (0, 0)),
            pl.BlockSpec((1, EH), lambda i: (0, 0)),
            pl.BlockSpec((1, HID), lambda i: (0, 0)),
        ],
        out_specs=[
            pl.BlockSpec((blk, HID), lambda i: (i, 0)),
            pl.BlockSpec((blk, HID), lambda i: (i, 0)),
            pl.BlockSpec((blk, 2), lambda i: (i, 0)),
            pl.BlockSpec((1, HID), lambda i: (0, 0)),
        ],
        out_shape=[
            jax.ShapeDtypeStruct((N, HID), jnp.float32),
            jax.ShapeDtypeStruct((N, HID), jnp.float32),
            jax.ShapeDtypeStruct((N, 2), jnp.float32),
            jax.ShapeDtypeStruct((1, HID), jnp.float32),
        ],
    )(h, lp['lin_l_W'], lp['lin_l_b'], lp['lin_r_W'], lp['lin_r_b'],
      lp['lin_e_W'], mean_e, att1)


def _ee_body(ea_ref, we_ref, o_ref):
    o_ref[...] = jnp.dot(ea_ref[...], we_ref[...],
                         preferred_element_type=jnp.float32)


def _ee_proj(ea_p, we):
    blk = 2048
    return pl.pallas_call(
        _ee_body,
        grid=(EPAD // blk,),
        in_specs=[
            pl.BlockSpec((blk, EH), lambda i: (i, 0)),
            pl.BlockSpec((EH, HID), lambda i: (0, 0)),
        ],
        out_specs=pl.BlockSpec((blk, HID), lambda i: (i, 0)),
        out_shape=jax.ShapeDtypeStruct((EPAD, HID), jnp.float32),
    )(ea_p, we)


def _combine_body(outc, sc, al, xl, h_ref, mrow, bias, ng, nb, o_ref):
    mv = mrow[0, 0]
    el = jnp.exp(al[...] - mv)
    num4 = outc[0] + outc[1]
    num = jnp.concatenate([num4[0], num4[1], num4[2], num4[3]], axis=1)
    scb = sc[...]
    s0 = scb[:, 0] + scb[:, 2] + el[:, 0]
    s1 = scb[:, 1] + scb[:, 3] + el[:, 1]
    xlb = xl[...]
    n0 = num[:, :64] + el[:, 0:1] * xlb[:, :64]
    n1 = num[:, 64:] + el[:, 1:2] * xlb[:, 64:]
    o0 = n0 / (s0[:, None] + 1e-16)
    o1 = n1 / (s1[:, None] + 1e-16)
    xn = jnp.concatenate([o0, o1], axis=1) + bias[...]
    mu = jnp.mean(xn, axis=-1, keepdims=True)
    v = jnp.mean((xn - mu) * (xn - mu), axis=-1, keepdims=True)
    xn = (xn - mu) * lax.rsqrt(v + 1e-5) * ng[...] + nb[...]
    o_ref[...] = jnp.maximum(xn, 0.0) + h_ref[...]


def _combine(outc, sc, al, xl, h, mrow, lp):
    blk = 1000
    return pl.pallas_call(
        _combine_body,
        grid=(N // blk,),
        in_specs=[
            pl.BlockSpec((NC, NQ, blk, CQ), lambda i: (0, 0, i, 0)),
            pl.BlockSpec((blk, 4), lambda i: (i, 0)),
            pl.BlockSpec((blk, 2), lambda i: (i, 0)),
            pl.BlockSpec((blk, HID), lambda i: (i, 0)),
            pl.BlockSpec((blk, HID), lambda i: (i, 0)),
            pl.BlockSpec((1, HID), lambda i: (0, 0)),
            pl.BlockSpec((HID,), lambda i: (0,)),
            pl.BlockSpec((HID,), lambda i: (0,)),
            pl.BlockSpec((HID,), lambda i: (0,)),
        ],
        out_specs=pl.BlockSpec((blk, HID), lambda i: (i, 0)),
        out_shape=jax.ShapeDtypeStruct((N, HID), jnp.float32),
    )(outc, sc, al, xl, h, mrow, lp['bias'], lp['ng'], lp['nb'])


def _pool_body(b_ref, h_ref, sums_o, cnt_o):
    bb = b_ref[...]
    gids = lax.broadcasted_iota(jnp.int32, (1, G), 1)
    onehot = (bb == gids).astype(jnp.float32)
    dn = (((0,), (0,)), ((), ()))

    @pl.when(pl.program_id(0) == 0)
    def _():
        sums_o[...] = jnp.zeros_like(sums_o)
        cnt_o[...] = jnp.zeros_like(cnt_o)

    sums_o[...] += lax.dot_general(onehot, h_ref[...], dn,
                                   preferred_element_type=jnp.float32)
    ones = jnp.ones((bb.shape[0], 8), jnp.float32)
    cnt_o[...] += lax.dot_general(onehot, ones, dn,
                                  preferred_element_type=jnp.float32)


def _pool(batch2, h):
    blk = 1000
    return pl.pallas_call(
        _pool_body,
        grid=(N // blk,),
        in_specs=[
            pl.BlockSpec((blk, 1), lambda i: (i, 0)),
            pl.BlockSpec((blk, HID), lambda i: (i, 0)),
        ],
        out_specs=[
            pl.BlockSpec((G, HID), lambda i: (0, 0)),
            pl.BlockSpec((G, 8), lambda i: (0, 0)),
        ],
        out_shape=[
            jax.ShapeDtypeStruct((G, HID), jnp.float32),
            jax.ShapeDtypeStruct((G, 8), jnp.float32),
        ],
    )(batch2, h)


def _heads_body(sums, cnts, wr, br,
                w1a, b1a, ga, bea, w2a, b2a, w3a, b3a,
                w1b, b1b, gb, beb, w2b, b2b, w3b, b3b,
                w1c, b1c, gc, bec, w2c, b2c, w3c, b3c,
                oa, ob, oc):
    gemb = sums[...] / jnp.maximum(cnts[:, :1], 1.0)
    gemb = jnp.tanh(jnp.dot(gemb, wr[...], preferred_element_type=jnp.float32)
                    + br[...])

    def head(w1, b1, g, be, w2, b2, w3, b3, o):
        h1 = jnp.dot(gemb, w1[...], preferred_element_type=jnp.float32) + b1[...]
        mu = jnp.mean(h1, axis=-1, keepdims=True)
        v = jnp.mean((h1 - mu) * (h1 - mu), axis=-1, keepdims=True)
        h1 = (h1 - mu) * lax.rsqrt(v + 1e-5) * g[...] + be[...]
        h1 = jnp.maximum(h1, 0.0)
        h2 = jnp.maximum(
            jnp.dot(h1, w2[...], preferred_element_type=jnp.float32) + b2[...], 0.0)
        o[...] = jnp.dot(h2, w3[...], preferred_element_type=jnp.float32) + b3[...]

    head(w1a, b1a, ga, bea, w2a, b2a, w3a, b3a, oa)
    head(w1b, b1b, gb, beb, w2b, b2b, w3b, b3b, ob)
    head(w1c, b1c, gc, bec, w2c, b2c, w3c, b3c, oc)


def _heads(sums, cnts, params):
    hp = params['heads']
    args = [sums, cnts, params['readout']['W'], params['readout']['b']]
    for t in ['DAT', 'NET', 'SERT']:
        p = hp[t]
        args += [p['W1'], p['b1'], p['g'], p['be'], p['W2'], p['b2'], p['W3'], p['b3']]
    full = lambda shape: pl.BlockSpec(shape, lambda: tuple(0 for _ in shape))
    in_specs = [full(a.shape) for a in args]
    return pl.pallas_call(
        _heads_body,
        grid=(),
        in_specs=in_specs,
        out_specs=[full((G, 3))] * 3,
        out_shape=[jax.ShapeDtypeStruct((G, 3), jnp.float32)] * 3,
    )(*args)


@functools.partial(
    pl.kernel,
    out_type=[
        jax.ShapeDtypeStruct((EPAD,), jnp.float32),
        jax.ShapeDtypeStruct((EPAD,), jnp.float32),
        jax.ShapeDtypeStruct((NW * 16,), jnp.float32),
    ],
    mesh=_mesh,
    compiler_params=pltpu.CompilerParams(needs_layout_passes=False,
                                         use_tc_tiling_on_sc=False),
    scratch_types=[
        pltpu.VMEM((WA,), jnp.int32),
        pltpu.VMEM((WA,), jnp.int32),
        pltpu.VMEM((WA, HID), jnp.float32),
        pltpu.VMEM((WA, HID), jnp.float32),
        pltpu.VMEM((WA, HID), jnp.float32),
        pltpu.VMEM((WA,), jnp.float32),
        pltpu.VMEM((WA,), jnp.float32),
        pltpu.VMEM((HID,), jnp.float32),
        pltpu.VMEM((16,), jnp.float32),
        pltpu.SemaphoreType.DMA,
    ],
)
def _sc_alpha(xl, xr, ee, srcp, dstp, att,
              a0_o, a1_o, tmax_o,
              srcv, dstv, xjv, xiv, eev, a0b, a1b, attv, tmb, sem):
    wid = _wid()
    wbase = wid * EPW
    pltpu.sync_copy(att, attv)
    iota16 = lax.iota(jnp.int32, 16)

    def win(w, tmaxv):
        g0 = wbase + w * WA
        pltpu.sync_copy(srcp.at[pl.ds(g0, WA)], srcv)
        pltpu.sync_copy(dstp.at[pl.ds(g0, WA)], dstv)
        cj = pltpu.async_copy(xl.at[srcv], xjv, sem)
        ci = pltpu.async_copy(xr.at[dstv], xiv, sem)
        ce = pltpu.async_copy(ee.at[pl.ds(g0, WA)], eev, sem)
        cj.wait()
        ci.wait()
        ce.wait()

        def group(g, tmaxv):
            rows = iota16 + g * 16

            def cbody(c, acc):
                cb = jnp.broadcast_to(c, (16,))
                z = (plsc.load_gather(xjv, [rows, cb]) +
                     plsc.load_gather(xiv, [rows, cb]) +
                     plsc.load_gather(eev, [rows, cb]))
                m = jnp.maximum(z, 0.2 * z)
                return acc + m * plsc.load_gather(attv, [cb])

            acc0 = lax.fori_loop(0, 64, cbody, jnp.zeros((16,), jnp.float32))
            acc1 = lax.fori_loop(64, HID, cbody, jnp.zeros((16,), jnp.float32))
            eidv = (g0 + g * 16) + iota16
            maskv = eidv < E
            acc0 = jnp.where(maskv, acc0, NEG)
            acc1 = jnp.where(maskv, acc1, NEG)
            a0b[pl.ds(g * 16, 16)] = acc0
            a1b[pl.ds(g * 16, 16)] = acc1
            return jnp.maximum(tmaxv, jnp.maximum(acc0, acc1))

        tmaxv = lax.fori_loop(0, WA // 16, group, tmaxv)
        pltpu.sync_copy(a0b, a0_o.at[pl.ds(g0, WA)])
        pltpu.sync_copy(a1b, a1_o.at[pl.ds(g0, WA)])
        return tmaxv

    tmaxv = lax.fori_loop(0, EPW // WA, win, jnp.full((16,), NEG, jnp.float32))
    tmb[...] = tmaxv
    pltpu.sync_copy(tmb, tmax_o.at[pl.ds(wid * 16, 16)])


@functools.partial(
    pl.kernel,
    out_type=[
        jax.ShapeDtypeStruct((NC, NQ, NP, CQ), jnp.float32),
        jax.ShapeDtypeStruct((NC * 2 * NP,), jnp.float32),
    ],
    mesh=_mesh,
    compiler_params=pltpu.CompilerParams(needs_layout_passes=False,
                                         use_tc_tiling_on_sc=False),
    scratch_types=[
        pltpu.VMEM((WA,), jnp.int32),
        pltpu.VMEM((WA,), jnp.int32),
        pltpu.VMEM((WA,), jnp.float32),
        pltpu.VMEM((WA,), jnp.float32),
        pltpu.VMEM((WA, CQ), jnp.float32),
        pltpu.VMEM((16,), jnp.float32),
        pltpu.VMEM_SHARED((NP, CQ), jnp.float32),
        pltpu.VMEM_SHARED((NP,), jnp.float32),
        pltpu.VMEM_SHARED((NP,), jnp.float32),
        pltpu.SemaphoreType.DMA,
    ],
)
def _sc_scatter(xl0, xl1, xl2, xl3, srcp, dstp, a0, a1, mvec, zacc, zn,
                outq_o, s_o,
                srcv, dstv, av, ev, xjv, mv, acc, s0sh, s1sh, sem):
    cid = lax.axis_index("c")
    sid = lax.axis_index("s")
    wid = sid * NC + cid
    pltpu.sync_copy(mvec, mv)
    mvv = mv[...]
    iota16 = lax.iota(jnp.int32, 16)
    xls = [xl0, xl1, xl2, xl3]

    for q in range(NQ):
        pltpu.sync_copy(zacc, acc.at[pl.ds(sid * RPT, RPT)])
        if q == 0 or q == 2:
            ssh = s0sh if q == 0 else s1sh

            @pl.when(sid == 0)
            def _():
                pltpu.sync_copy(zn, ssh)

        plsc.subcore_barrier()
        aref = a0 if q < 2 else a1

        def win(w, _):
            g0 = wid * EPW + w * WA
            pltpu.sync_copy(srcp.at[pl.ds(g0, WA)], srcv)
            pltpu.sync_copy(dstp.at[pl.ds(g0, WA)], dstv)
            pltpu.sync_copy(aref.at[pl.ds(g0, WA)], av)
            pltpu.async_copy(xls[q].at[srcv], xjv, sem).wait()
            for t in range(WA // 16):
                sl = pl.ds(t * 16, 16)
                ev[sl] = jnp.exp(av[sl] - mvv)
            if q == 0 or q == 2:
                pltpu.sync_copy(ev, ssh.at[dstv], add=True)
            for t in range(WA // 16):
                rows = iota16 + t * 16
                evt = ev[pl.ds(t * 16, 16)]
                for c in range(CQ):
                    cb = jnp.full((16,), c, jnp.int32)
                    v = plsc.load_gather(xjv, [rows, cb]) * evt
                    plsc.store_scatter(xjv, [rows, cb], v)
            pltpu.sync_copy(xjv, acc.at[dstv], add=True)
            return 0

        lax.fori_loop(0, EPW // WA, win, 0)
        plsc.subcore_barrier()
        pltpu.sync_copy(acc.at[pl.ds(sid * RPT, RPT)],
                        outq_o.at[cid, q, pl.ds(sid * RPT, RPT)])
        if q == 0 or q == 2:
            hsel = 0 if q == 0 else 1

            @pl.when(sid == 0)
            def _():
                pltpu.sync_copy(ssh, s_o.at[pl.ds(cid * (2 * NP)
                                                  + hsel * NP, NP)])
        plsc.subcore_barrier()


def kernel(x, edge_attr, edge_index, batch, params):
    src = edge_index[0]
    dst = edge_index[1]
    padi = jnp.arange(PADW, dtype=jnp.int32)
    srcp = jnp.concatenate([src, padi])
    dstp = jnp.concatenate([dst, padi])

    h = _node_encoder(x, params['node'])
    ea, mean_e = _edge_encoder(edge_attr, params['edge'])
    ea_p = jnp.pad(ea, ((0, PADW), (0, 0)))

    zacc = jnp.zeros((RPT, CQ), jnp.float32)
    zn = jnp.zeros((NP,), jnp.float32)

    for lp in params['layers']:
        att1 = lp['att'].reshape(1, HID)
        xl, xr, al, amax = _pre_layer(h, mean_e, att1, lp)
        ee = _ee_proj(ea_p, lp['lin_e_W'])
        a0, a1, tmax = _sc_alpha(xl, xr, ee, srcp, dstp, lp['att'].reshape(HID))
        M = jnp.maximum(jnp.max(tmax), amax[0, 0])
        mvec = jnp.full((16,), M, jnp.float32)
        xls = jnp.moveaxis(xl.reshape(N, NQ, CQ), 1, 0)
        outq, s = _sc_scatter(xls[0], xls[1], xls[2], xls[3], srcp, dstp,
                              a0, a1, mvec, zacc, zn)
        sc = jnp.moveaxis(s.reshape(NC, 2, NP)[:, :, :N], 2, 0).reshape(N, 4)
        mrow = jnp.full((1, HID), M, jnp.float32)
        h = _combine(outq, sc, al, xl, h, mrow, lp)

    sums, cnt2 = _pool(batch.reshape(N, 1), h)
    return tuple(_heads(sums, cnt2, params))


# speedup vs baseline: 9.4409x; 1.1301x over previous
import functools

import jax
import jax.numpy as jnp
from jax import lax
from jax.experimental import pallas as pl
from jax.experimental.pallas import tpu as pltpu
from jax.experimental.pallas import tpu_sc as plsc

N = 50000
E = 800000
G = 128
ND = 86
ED = 18
HID = 128
EH = 64

NC = 2
NS = 16
NW = NC * NS
EPW = 25600
EPAD = NW * EPW
PADW = EPAD - E
NEG = -3e38

NQ = 4
CQ = HID // NQ
NP = 50048
RPT = NP // NS

WA = 128
CW = EPW // WA
SR = 8

_mesh = plsc.VectorSubcoreMesh(core_axis_name="c", subcore_axis_name="s")


def _wid():
    return lax.axis_index("s") * NC + lax.axis_index("c")


def _node_enc_body(x_ref, w_ref, b_ref, g_ref, be_ref, o_ref):
    h = jnp.dot(x_ref[...], w_ref[...], preferred_element_type=jnp.float32)
    h = h + b_ref[...]
    mu = jnp.mean(h, axis=-1, keepdims=True)
    v = jnp.mean((h - mu) * (h - mu), axis=-1, keepdims=True)
    h = (h - mu) * lax.rsqrt(v + 1e-5) * g_ref[...] + be_ref[...]
    o_ref[...] = jnp.maximum(h, 0.0)


def _node_encoder(x, p):
    blk = 1000
    return pl.pallas_call(
        _node_enc_body,
        grid=(N // blk,),
        in_specs=[
            pl.BlockSpec((blk, ND), lambda i: (i, 0)),
            pl.BlockSpec((ND, HID), lambda i: (0, 0)),
            pl.BlockSpec((HID,), lambda i: (0,)),
            pl.BlockSpec((HID,), lambda i: (0,)),
            pl.BlockSpec((HID,), lambda i: (0,)),
        ],
        out_specs=pl.BlockSpec((blk, HID), lambda i: (i, 0)),
        out_shape=jax.ShapeDtypeStruct((N, HID), jnp.float32),
    )(x, p['W'], p['b'], p['g'], p['be'])


def _edge_enc_body(ea_ref, w_ref, b_ref, g_ref, be_ref, o_ref, sum_ref):
    h = jnp.dot(ea_ref[...], w_ref[...], preferred_element_type=jnp.float32)
    h = h + b_ref[...]
    mu = jnp.mean(h, axis=-1, keepdims=True)
    v = jnp.mean((h - mu) * (h - mu), axis=-1, keepdims=True)
    h = (h - mu) * lax.rsqrt(v + 1e-5) * g_ref[...] + be_ref[...]
    h = jnp.maximum(h, 0.0)
    o_ref[...] = h

    @pl.when(pl.program_id(0) == 0)
    def _():
        sum_ref[...] = jnp.zeros_like(sum_ref)

    sum_ref[...] += jnp.sum(h, axis=0, keepdims=True)


def _edge_encoder(ea, p):
    blk = 2000
    out, colsum = pl.pallas_call(
        _edge_enc_body,
        grid=(E // blk,),
        in_specs=[
            pl.BlockSpec((blk, ED), lambda i: (i, 0)),
            pl.BlockSpec((ED, EH), lambda i: (0, 0)),
            pl.BlockSpec((EH,), lambda i: (0,)),
            pl.BlockSpec((EH,), lambda i: (0,)),
            pl.BlockSpec((EH,), lambda i: (0,)),
        ],
        out_specs=[
            pl.BlockSpec((blk, EH), lambda i: (i, 0)),
            pl.BlockSpec((1, EH), lambda i: (0, 0)),
        ],
        out_shape=[
            jax.ShapeDtypeStruct((E, EH), jnp.float32),
            jax.ShapeDtypeStruct((1, EH), jnp.float32),
        ],
    )(ea, p['W'], p['b'], p['g'], p['be'])
    return out, colsum / E


def _pre_body(h_ref, wl, bl, wr, br, we, me, att, xl_o, xr_o, al_o, amax_o):
    hb = h_ref[...]
    xl = jnp.dot(hb, wl[...], preferred_element_type=jnp.float32) + bl[...]
    xr = jnp.dot(hb, wr[...], preferred_element_type=jnp.float32) + br[...]
    eel = jnp.dot(me[...], we[...], preferred_element_type=jnp.float32)
    z = xl + xr + eel
    m = jnp.maximum(z, 0.2 * z) * att[...]
    a0 = jnp.sum(m[:, :64], axis=1, keepdims=True)
    a1 = jnp.sum(m[:, 64:], axis=1, keepdims=True)
    al = jnp.concatenate([a0, a1], axis=1)
    xl_o[...] = xl
    xr_o[...] = xr
    al_o[...] = al

    @pl.when(pl.program_id(0) == 0)
    def _():
        amax_o[...] = jnp.full_like(amax_o, NEG)

    amax_o[...] = jnp.maximum(amax_o[...], jnp.max(al))


def _pre_layer(h, mean_e, att1, lp):
    blk = 1000
    return pl.pallas_call(
        _pre_body,
        grid=(N // blk,),
        in_specs=[
            pl.BlockSpec((blk, HID), lambda i: (i, 0)),
            pl.BlockSpec((HID, HID), lambda i: (0, 0)),
            pl.BlockSpec((HID,), lambda i: (0,)),
            pl.BlockSpec((HID, HID), lambda i: (0, 0)),
            pl.BlockSpec((HID,), lambda i: (0,)),
            pl.BlockSpec((EH, HID), lambda i: (0, 0)),
            pl.BlockSpec((1, EH), lambda i: (0, 0)),
            pl.BlockSpec((1, HID), lambda i: (0, 0)),
        ],
        out_specs=[
            pl.BlockSpec((blk, HID), lambda i: (i, 0)),
            pl.BlockSpec((blk, HID), lambda i: (i, 0)),
            pl.BlockSpec((blk, 2), lambda i: (i, 0)),
            pl.BlockSpec((1, HID), lambda i: (0, 0)),
        ],
        out_shape=[
            jax.ShapeDtypeStruct((N, HID), jnp.float32),
            jax.ShapeDtypeStruct((N, HID), jnp.float32),
            jax.ShapeDtypeStruct((N, 2), jnp.float32),
            jax.ShapeDtypeStruct((1, HID), jnp.float32),
        ],
    )(h, lp['lin_l_W'], lp['lin_l_b'], lp['lin_r_W'], lp['lin_r_b'],
      lp['lin_e_W'], mean_e, att1)


def _ee_body(ea_ref, we_ref, o_ref):
    o_ref[...] = jnp.dot(ea_ref[...], we_ref[...],
                         preferred_element_type=jnp.float32)


def _ee_proj(ea_p, we):
    blk = 2048
    return pl.pallas_call(
        _ee_body,
        grid=(EPAD // blk,),
        in_specs=[
            pl.BlockSpec((blk, EH), lambda i: (i, 0)),
            pl.BlockSpec((EH, HID), lambda i: (0, 0)),
        ],
        out_specs=pl.BlockSpec((blk, HID), lambda i: (i, 0)),
        out_shape=jax.ShapeDtypeStruct((EPAD, HID), jnp.float32),
    )(ea_p, we)


def _combine_body(outc, sc, al, xl, h_ref, mrow, bias, ng, nb, o_ref):
    mv = mrow[0, 0]
    el = jnp.exp(al[...] - mv)
    num4 = outc[0] + outc[1]
    num = jnp.concatenate([num4[0], num4[1], num4[2], num4[3]], axis=1)
    scb = sc[...]
    s0 = scb[:, 0] + scb[:, 2] + el[:, 0]
    s1 = scb[:, 1] + scb[:, 3] + el[:, 1]
    xlb = xl[...]
    n0 = num[:, :64] + el[:, 0:1] * xlb[:, :64]
    n1 = num[:, 64:] + el[:, 1:2] * xlb[:, 64:]
    o0 = n0 / (s0[:, None] + 1e-16)
    o1 = n1 / (s1[:, None] + 1e-16)
    xn = jnp.concatenate([o0, o1], axis=1) + bias[...]
    mu = jnp.mean(xn, axis=-1, keepdims=True)
    v = jnp.mean((xn - mu) * (xn - mu), axis=-1, keepdims=True)
    xn = (xn - mu) * lax.rsqrt(v + 1e-5) * ng[...] + nb[...]
    o_ref[...] = jnp.maximum(xn, 0.0) + h_ref[...]


def _combine(outc, sc, al, xl, h, mrow, lp):
    blk = 1000
    return pl.pallas_call(
        _combine_body,
        grid=(N // blk,),
        in_specs=[
            pl.BlockSpec((NC, NQ, blk, CQ), lambda i: (0, 0, i, 0)),
            pl.BlockSpec((blk, 4), lambda i: (i, 0)),
            pl.BlockSpec((blk, 2), lambda i: (i, 0)),
            pl.BlockSpec((blk, HID), lambda i: (i, 0)),
            pl.BlockSpec((blk, HID), lambda i: (i, 0)),
            pl.BlockSpec((1, HID), lambda i: (0, 0)),
            pl.BlockSpec((HID,), lambda i: (0,)),
            pl.BlockSpec((HID,), lambda i: (0,)),
            pl.BlockSpec((HID,), lambda i: (0,)),
        ],
        out_specs=pl.BlockSpec((blk, HID), lambda i: (i, 0)),
        out_shape=jax.ShapeDtypeStruct((N, HID), jnp.float32),
    )(outc, sc, al, xl, h, mrow, lp['bias'], lp['ng'], lp['nb'])


def _pool_body(b_ref, h_ref, sums_o, cnt_o):
    bb = b_ref[...]
    gids = lax.broadcasted_iota(jnp.int32, (1, G), 1)
    onehot = (bb == gids).astype(jnp.float32)
    dn = (((0,), (0,)), ((), ()))

    @pl.when(pl.program_id(0) == 0)
    def _():
        sums_o[...] = jnp.zeros_like(sums_o)
        cnt_o[...] = jnp.zeros_like(cnt_o)

    sums_o[...] += lax.dot_general(onehot, h_ref[...], dn,
                                   preferred_element_type=jnp.float32)
    ones = jnp.ones((bb.shape[0], 8), jnp.float32)
    cnt_o[...] += lax.dot_general(onehot, ones, dn,
                                  preferred_element_type=jnp.float32)


def _pool(batch2, h):
    blk = 1000
    return pl.pallas_call(
        _pool_body,
        grid=(N // blk,),
        in_specs=[
            pl.BlockSpec((blk, 1), lambda i: (i, 0)),
            pl.BlockSpec((blk, HID), lambda i: (i, 0)),
        ],
        out_specs=[
            pl.BlockSpec((G, HID), lambda i: (0, 0)),
            pl.BlockSpec((G, 8), lambda i: (0, 0)),
        ],
        out_shape=[
            jax.ShapeDtypeStruct((G, HID), jnp.float32),
            jax.ShapeDtypeStruct((G, 8), jnp.float32),
        ],
    )(batch2, h)


def _heads_body(sums, cnts, wr, br,
                w1a, b1a, ga, bea, w2a, b2a, w3a, b3a,
                w1b, b1b, gb, beb, w2b, b2b, w3b, b3b,
                w1c, b1c, gc, bec, w2c, b2c, w3c, b3c,
                oa, ob, oc):
    gemb = sums[...] / jnp.maximum(cnts[:, :1], 1.0)
    gemb = jnp.tanh(jnp.dot(gemb, wr[...], preferred_element_type=jnp.float32)
                    + br[...])

    def head(w1, b1, g, be, w2, b2, w3, b3, o):
        h1 = jnp.dot(gemb, w1[...], preferred_element_type=jnp.float32) + b1[...]
        mu = jnp.mean(h1, axis=-1, keepdims=True)
        v = jnp.mean((h1 - mu) * (h1 - mu), axis=-1, keepdims=True)
        h1 = (h1 - mu) * lax.rsqrt(v + 1e-5) * g[...] + be[...]
        h1 = jnp.maximum(h1, 0.0)
        h2 = jnp.maximum(
            jnp.dot(h1, w2[...], preferred_element_type=jnp.float32) + b2[...], 0.0)
        o[...] = jnp.dot(h2, w3[...], preferred_element_type=jnp.float32) + b3[...]

    head(w1a, b1a, ga, bea, w2a, b2a, w3a, b3a, oa)
    head(w1b, b1b, gb, beb, w2b, b2b, w3b, b3b, ob)
    head(w1c, b1c, gc, bec, w2c, b2c, w3c, b3c, oc)


def _heads(sums, cnts, params):
    hp = params['heads']
    args = [sums, cnts, params['readout']['W'], params['readout']['b']]
    for t in ['DAT', 'NET', 'SERT']:
        p = hp[t]
        args += [p['W1'], p['b1'], p['g'], p['be'], p['W2'], p['b2'], p['W3'], p['b3']]
    full = lambda shape: pl.BlockSpec(shape, lambda: tuple(0 for _ in shape))
    in_specs = [full(a.shape) for a in args]
    return pl.pallas_call(
        _heads_body,
        grid=(),
        in_specs=in_specs,
        out_specs=[full((G, 3))] * 3,
        out_shape=[jax.ShapeDtypeStruct((G, 3), jnp.float32)] * 3,
    )(*args)


@functools.partial(
    pl.kernel,
    out_type=[
        jax.ShapeDtypeStruct((EPAD,), jnp.float32),
        jax.ShapeDtypeStruct((EPAD,), jnp.float32),
        jax.ShapeDtypeStruct((NW * 16,), jnp.float32),
    ],
    mesh=_mesh,
    compiler_params=pltpu.CompilerParams(needs_layout_passes=False,
                                         use_tc_tiling_on_sc=False),
    scratch_types=[
        pltpu.VMEM((WA,), jnp.int32),
        pltpu.VMEM((WA,), jnp.int32),
        pltpu.VMEM((WA, HID), jnp.float32),
        pltpu.VMEM((WA, HID), jnp.float32),
        pltpu.VMEM((WA, HID), jnp.float32),
        pltpu.VMEM((WA,), jnp.float32),
        pltpu.VMEM((WA,), jnp.float32),
        pltpu.VMEM((HID,), jnp.float32),
        pltpu.VMEM((16,), jnp.float32),
        pltpu.SemaphoreType.DMA,
    ],
)
def _sc_alpha(xl, xr, ee, srcp, dstp, att,
              a0_o, a1_o, tmax_o,
              srcv, dstv, xjv, xiv, eev, a0b, a1b, attv, tmb, sem):
    wid = _wid()
    wbase = wid * EPW
    pltpu.sync_copy(att, attv)
    iota16 = lax.iota(jnp.int32, 16)

    def win(w, tmaxv):
        g0 = wbase + w * WA
        pltpu.sync_copy(srcp.at[pl.ds(g0, WA)], srcv)
        pltpu.sync_copy(dstp.at[pl.ds(g0, WA)], dstv)
        cj = pltpu.async_copy(xl.at[srcv], xjv, sem)
        ci = pltpu.async_copy(xr.at[dstv], xiv, sem)
        ce = pltpu.async_copy(ee.at[pl.ds(g0, WA)], eev, sem)
        cj.wait()
        ci.wait()
        ce.wait()

        def group(g, tmaxv):
            rows = iota16 + g * 16

            def quad(k, ab):
                pa, pb = ab
                c0 = k * 4
                ms = []
                for d in range(4):
                    cb = jnp.broadcast_to(c0 + d, (16,))
                    z = (plsc.load_gather(xjv, [rows, cb]) +
                         plsc.load_gather(xiv, [rows, cb]) +
                         plsc.load_gather(eev, [rows, cb]))
                    m = jnp.maximum(z, 0.2 * z)
                    ms.append(m * plsc.load_gather(attv, [cb]))
                return (pa + (ms[0] + ms[1]), pb + (ms[2] + ms[3]))

            z16 = jnp.zeros((16,), jnp.float32)
            pa, pb = lax.fori_loop(0, 16, quad, (z16, z16))
            acc0 = pa + pb
            pa, pb = lax.fori_loop(16, 32, quad, (z16, z16))
            acc1 = pa + pb
            eidv = (g0 + g * 16) + iota16
            maskv = eidv < E
            acc0 = jnp.where(maskv, acc0, NEG)
            acc1 = jnp.where(maskv, acc1, NEG)
            a0b[pl.ds(g * 16, 16)] = acc0
            a1b[pl.ds(g * 16, 16)] = acc1
            return jnp.maximum(tmaxv, jnp.maximum(acc0, acc1))

        tmaxv = lax.fori_loop(0, WA // 16, group, tmaxv)
        pltpu.sync_copy(a0b, a0_o.at[pl.ds(g0, WA)])
        pltpu.sync_copy(a1b, a1_o.at[pl.ds(g0, WA)])
        return tmaxv

    tmaxv = lax.fori_loop(0, EPW // WA, win, jnp.full((16,), NEG, jnp.float32))
    tmb[...] = tmaxv
    pltpu.sync_copy(tmb, tmax_o.at[pl.ds(wid * 16, 16)])


@functools.partial(
    pl.kernel,
    out_type=[
        jax.ShapeDtypeStruct((NC, NQ, NP, CQ), jnp.float32),
        jax.ShapeDtypeStruct((NC * 2 * NP,), jnp.float32),
    ],
    mesh=_mesh,
    compiler_params=pltpu.CompilerParams(needs_layout_passes=False,
                                         use_tc_tiling_on_sc=False),
    scratch_types=[
        pltpu.VMEM((SR, WA), jnp.int32),
        pltpu.VMEM((SR, WA), jnp.int32),
        pltpu.VMEM((SR, WA), jnp.float32),
        pltpu.VMEM((WA, CQ), jnp.float32),
        pltpu.VMEM((WA, CQ), jnp.float32),
        pltpu.VMEM((16,), jnp.float32),
        pltpu.VMEM_SHARED((NP, CQ), jnp.float32),
        pltpu.VMEM_SHARED((NP,), jnp.float32),
        pltpu.VMEM_SHARED((NP,), jnp.float32),
        pltpu.SemaphoreType.DMA,
        pltpu.SemaphoreType.DMA,
        pltpu.SemaphoreType.DMA,
        pltpu.SemaphoreType.DMA,
        pltpu.SemaphoreType.DMA,
    ],
)
def _sc_scatter(xl0, xl1, xl2, xl3, srcp2, dstp2, a02, a12, mvec, zacc, zn,
                outq_o, s_o,
                srcv2, dstv2, ev2, b0, b1, mv, acc, s0sh, s1sh,
                g0s, g1s, s0s, s1s, esem):
    cid = lax.axis_index("c")
    sid = lax.axis_index("s")
    wid = sid * NC + cid
    pltpu.sync_copy(mvec, mv)
    mvv = mv[...]
    iota16 = lax.iota(jnp.int32, 16)
    xls = [xl0, xl1, xl2, xl3]
    bufs = [b0, b1]
    gsem = [g0s, g1s]
    ssem = [s0s, s1s]
    r0 = wid * CW

    for q in range(NQ):
        pltpu.sync_copy(zacc, acc.at[pl.ds(sid * RPT, RPT)])
        if q == 0 or q == 2:
            ssh = s0sh if q == 0 else s1sh
            aref2 = a02 if q == 0 else a12

            @pl.when(sid == 0)
            def _():
                pltpu.sync_copy(zn, ssh)

        plsc.subcore_barrier()

        def win(w, _):
            rw = r0 + w * SR
            pltpu.sync_copy(srcp2.at[pl.ds(rw, SR)], srcv2)
            pltpu.sync_copy(dstp2.at[pl.ds(rw, SR)], dstv2)
            if q == 0 or q == 2:
                pltpu.sync_copy(aref2.at[pl.ds(rw, SR)], ev2)

                def erow(r, _):
                    rb = jnp.broadcast_to(r, (16,))
                    for t in range(WA // 16):
                        cols = iota16 + t * 16
                        v = plsc.load_gather(ev2, [rb, cols])
                        plsc.store_scatter(ev2, [rb, cols],
                                           jnp.exp(v - mvv))
                    return 0

                lax.fori_loop(0, SR, erow, 0)

            for b in range(2):
                pltpu.async_copy(xls[q].at[srcv2.at[b]], bufs[b], gsem[b])

            def grp(g, _):
                jts = [g * 2 + b for b in range(2)]
                for b in range(2):
                    jt = jts[b]
                    jb = jnp.broadcast_to(jt, (16,))
                    pltpu.make_async_copy(xls[q].at[srcv2.at[jt]], bufs[b],
                                          gsem[b]).wait()
                    buf = bufs[b]

                    def tloop(t, _):
                        evt = plsc.load_gather(ev2, [jb, iota16 + t * 16])
                        rows = iota16 + t * 16
                        for c in range(CQ):
                            cb = jnp.full((16,), c, jnp.int32)
                            v = plsc.load_gather(buf, [rows, cb]) * evt
                            plsc.store_scatter(buf, [rows, cb], v)
                        return 0

                    lax.fori_loop(0, WA // 16, tloop, 0)
                    if q == 0 or q == 2:
                        pltpu.async_copy(ev2.at[jt], ssh.at[dstv2.at[jt]],
                                         esem, add=True)
                    pltpu.async_copy(bufs[b], acc.at[dstv2.at[jt]],
                                     ssem[b], add=True)
                for b in range(2):
                    pltpu.make_async_copy(bufs[b], acc.at[dstv2.at[0]],
                                          ssem[b]).wait()

                @pl.when(g < (SR // 2) - 1)
                def _():
                    for b in range(2):
                        pltpu.async_copy(xls[q].at[srcv2.at[jts[b] + 2]],
                                         bufs[b], gsem[b])

                return 0

            lax.fori_loop(0, SR // 2, grp, 0)
            if q == 0 or q == 2:
                pltpu.make_async_copy(srcp2.at[pl.ds(rw, SR)], dstv2,
                                      esem).wait()
            return 0

        lax.fori_loop(0, CW // SR, win, 0)
        plsc.subcore_barrier()
        pltpu.sync_copy(acc.at[pl.ds(sid * RPT, RPT)],
                        outq_o.at[cid, q, pl.ds(sid * RPT, RPT)])
        if q == 0 or q == 2:
            hsel = 0 if q == 0 else 1

            @pl.when(sid == 0)
            def _():
                pltpu.sync_copy(ssh, s_o.at[pl.ds(cid * (2 * NP)
                                                  + hsel * NP, NP)])
        plsc.subcore_barrier()


def kernel(x, edge_attr, edge_index, batch, params):
    src = edge_index[0]
    dst = edge_index[1]
    padi = jnp.arange(PADW, dtype=jnp.int32)
    srcp = jnp.concatenate([src, padi])
    dstp = jnp.concatenate([dst, padi])

    h = _node_encoder(x, params['node'])
    ea, mean_e = _edge_encoder(edge_attr, params['edge'])
    ea_p = jnp.pad(ea, ((0, PADW), (0, 0)))

    zacc = jnp.zeros((RPT, CQ), jnp.float32)
    zn = jnp.zeros((NP,), jnp.float32)

    for lp in params['layers']:
        att1 = lp['att'].reshape(1, HID)
        xl, xr, al, amax = _pre_layer(h, mean_e, att1, lp)
        ee = _ee_proj(ea_p, lp['lin_e_W'])
        a0, a1, tmax = _sc_alpha(xl, xr, ee, srcp, dstp, lp['att'].reshape(HID))
        M = jnp.maximum(jnp.max(tmax), amax[0, 0])
        mvec = jnp.full((16,), M, jnp.float32)
        xls = jnp.moveaxis(xl.reshape(N, NQ, CQ), 1, 0)
        outq, s = _sc_scatter(xls[0], xls[1], xls[2], xls[3],
                              srcp.reshape(EPAD // WA, WA),
                              dstp.reshape(EPAD // WA, WA),
                              a0.reshape(EPAD // WA, WA),
                              a1.reshape(EPAD // WA, WA), mvec, zacc, zn)
        sc = jnp.moveaxis(s.reshape(NC, 2, NP)[:, :, :N], 2, 0).reshape(N, 4)
        mrow = jnp.full((1, HID), M, jnp.float32)
        h = _combine(outq, sc, al, xl, h, mrow, lp)

    sums, cnt2 = _pool(batch.reshape(N, 1), h)
    return tuple(_heads(sums, cnt2, params))
